# Initial kernel scaffold; baseline (speedup 1.0000x reference)
import jax
import jax.numpy as jnp
from jax.experimental import pallas as pl


def kernel(score_pred, reg_pred, anchors, gts):
    raise NotImplementedError("write your pallas kernel here")


# speedup vs baseline: 115.9846x; 115.9846x over previous
import functools

import jax
import jax.numpy as jnp
from jax import lax
from jax.experimental import pallas as pl
from jax.experimental.pallas import tpu as pltpu

_N = 20000
_G = 50
_ROWS = 160
_LANES = 128
_NPAD = _ROWS * _LANES

_SAMPLE_MAX = 256
_NEG_THR = 0.3
_POS_THR = 0.5
_NP_RATE = 0.5
_POS_UPPER = int(_SAMPLE_MAX * _NP_RATE)


def _atl_kernel(a_ref, s_ref, r_ref, gts_ref, out_ref):
    f32 = jnp.float32
    a1 = a_ref[0]
    a2 = a_ref[1]
    a3 = a_ref[2]
    a4 = a_ref[3]
    area_a = (a3 - a1) * (a4 - a2)

    row_i = lax.broadcasted_iota(jnp.int32, (_ROWS, _LANES), 0)
    col_i = lax.broadcasted_iota(jnp.int32, (_ROWS, _LANES), 1)
    idx = row_i * _LANES + col_i
    idxf = idx.astype(f32)
    valid = idx < _N

    neg_one = jnp.full((_ROWS, _LANES), -1.0, f32)
    zeros = jnp.zeros((_ROWS, _LANES), f32)

    def body(g, carry):
        run_max, is_gtmax, mx1, my1, mx2, my2 = carry
        gx1 = gts_ref[g, 0]
        gy1 = gts_ref[g, 1]
        gx2 = gts_ref[g, 2]
        gy2 = gts_ref[g, 3]
        garea = (gx2 - gx1) * (gy2 - gy1)
        w = jnp.maximum(jnp.minimum(a3, gx2) - jnp.maximum(a1, gx1), 0.0)
        h = jnp.maximum(jnp.minimum(a4, gy2) - jnp.maximum(a2, gy1), 0.0)
        inter = w * h
        union = area_a + garea - inter
        iou = jnp.where(valid, inter / union, -1.0)
        cmax = jnp.max(iou)
        gidx = jnp.min(jnp.where(iou == cmax, idxf, 3e7))
        is_gtmax = jnp.maximum(is_gtmax, (idxf == gidx).astype(jnp.float32))
        better = iou > run_max
        run_max = jnp.where(better, iou, run_max)
        mx1 = jnp.where(better, gx1, mx1)
        my1 = jnp.where(better, gy1, my1)
        mx2 = jnp.where(better, gx2, mx2)
        my2 = jnp.where(better, gy2, my2)
        return run_max, is_gtmax, mx1, my1, mx2, my2

    init = (neg_one, zeros, zeros, zeros, zeros + 1.0, zeros + 1.0)
    run_max, is_gtmax_f, mx1, my1, mx2, my2 = lax.fori_loop(0, _G, body, init)
    is_gtmax = is_gtmax_f > 0.0

    pos_mask = jnp.logical_and(
        valid, jnp.logical_or(run_max > _POS_THR, is_gtmax))
    neg_mask = jnp.logical_and(
        valid, jnp.logical_and(run_max < _NEG_THR,
                               jnp.logical_not(is_gtmax)))
    pos_f = pos_mask.astype(f32)
    neg_f = neg_mask.astype(f32)
    p_full = jnp.sum(pos_f)
    q_full = jnp.sum(neg_f)
    k_pos = jnp.minimum(p_full, float(_POS_UPPER))
    score_bug = p_full >= float(_POS_UPPER)
    neg_bound = jnp.where(p_full < float(_POS_UPPER), p_full,
                          float(_POS_UPPER))
    q_eff = jnp.minimum(neg_bound, q_full)

    bf16 = jnp.bfloat16
    t_lane = (lax.broadcasted_iota(jnp.int32, (_LANES, _LANES), 0)
              <= lax.broadcasted_iota(jnp.int32, (_LANES, _LANES), 1)
              ).astype(bf16)
    t_row = (lax.broadcasted_iota(jnp.int32, (_ROWS, _ROWS), 0)
             > lax.broadcasted_iota(jnp.int32, (_ROWS, _ROWS), 1)
             ).astype(bf16)

    def prefix_incl(m_f32):
        incl = jnp.dot(m_f32.astype(bf16), t_lane,
                       preferred_element_type=f32)
        rowtot = jnp.broadcast_to(
            jnp.sum(m_f32, axis=1, keepdims=True), (_ROWS, _LANES))
        offs = jnp.dot(t_row, rowtot.astype(bf16),
                       preferred_element_type=f32)
        return incl + offs

    sel_pos = jnp.logical_and(pos_mask, prefix_incl(pos_f) <= k_pos)
    sel_neg = jnp.logical_and(neg_mask, prefix_incl(neg_f) <= q_eff)
    sel_pos_f = sel_pos.astype(f32)

    s0 = s_ref[0]
    s1 = s_ref[1]
    sm = jnp.maximum(s0, s1)
    lse = sm + jnp.log(jnp.exp(s0 - sm) + jnp.exp(s1 - sm))
    logp0_score = s0 - lse
    logp1_score = s1 - lse
    am = jnp.maximum(jnp.maximum(a1, a2), jnp.maximum(a3, a4))
    alse = am + jnp.log(jnp.exp(a1 - am) + jnp.exp(a2 - am)
                        + jnp.exp(a3 - am) + jnp.exp(a4 - am))
    logp0_anch = a1 - alse
    pos_term = jnp.where(score_bug, logp0_anch, logp0_score)

    aw = a3 - a1
    ah = a4 - a2
    acx = a1 + aw * 0.5
    acy = a2 + ah * 0.5
    gw = mx2 - mx1
    gh = my2 - my1
    gcx = mx1 + gw * 0.5
    gcy = my1 + gh * 0.5
    tx = (gcx - acx) / aw
    ty = (gcy - acy) / ah
    tw = jnp.log(gw / aw)
    th = jnp.log(gh / ah)

    def sl1(pred, tgt):
        d = pred - tgt
        ad = jnp.abs(d)
        return jnp.where(ad < 1.0, 0.5 * d * d, ad - 0.5)

    reg_sum = (sl1(r_ref[0], tx) + sl1(r_ref[1], ty)
               + sl1(r_ref[2], tw) + sl1(r_ref[3], th))

    pos_cls = -jnp.sum(sel_pos_f * pos_term) / k_pos
    pos_reg = jnp.sum(sel_pos_f * reg_sum) / k_pos
    neg_cls = -jnp.sum(jnp.where(sel_neg, logp1_score, 0.0)) / q_eff
    out_ref[0, 0] = pos_cls + pos_reg + neg_cls


@jax.jit
def kernel(score_pred, reg_pred, anchors, gts):
    pad = _NPAD - _N
    a_t = jnp.pad(anchors.T, ((0, 0), (0, pad)))
    a_t = a_t.at[2:, _N:].set(1.0)
    s_t = jnp.pad(score_pred.T, ((0, 0), (0, pad)))
    r_t = jnp.pad(reg_pred.T, ((0, 0), (0, pad)))
    a3 = a_t.reshape(4, _ROWS, _LANES)
    s3 = s_t.reshape(2, _ROWS, _LANES)
    r3 = r_t.reshape(4, _ROWS, _LANES)

    out = pl.pallas_call(
        _atl_kernel,
        out_shape=jax.ShapeDtypeStruct((1, 1), jnp.float32),
        in_specs=[
            pl.BlockSpec(memory_space=pltpu.VMEM),
            pl.BlockSpec(memory_space=pltpu.VMEM),
            pl.BlockSpec(memory_space=pltpu.VMEM),
            pl.BlockSpec(memory_space=pltpu.SMEM),
        ],
        out_specs=pl.BlockSpec(memory_space=pltpu.SMEM),
    )(a3, s3, r3, gts)
    return out[0, 0]


# speedup vs baseline: 186.8901x; 1.6113x over previous
import functools

import jax
import jax.numpy as jnp
from jax import lax
from jax.experimental import pallas as pl
from jax.experimental.pallas import tpu as pltpu

_N = 20000
_G = 50
_ROWS = 160
_LANES = 128
_NPAD = _ROWS * _LANES

_SAMPLE_MAX = 256
_NEG_THR = 0.3
_POS_THR = 0.5
_NP_RATE = 0.5
_POS_UPPER = int(_SAMPLE_MAX * _NP_RATE)


def _atl_kernel(a_ref, s_ref, r_ref, gts_ref, out_ref, iou_s, cm_s, pm_s):
    f32 = jnp.float32
    a1 = a_ref[0]
    a2 = a_ref[1]
    a3 = a_ref[2]
    a4 = a_ref[3]
    area_a = (a3 - a1) * (a4 - a2)

    row_i = lax.broadcasted_iota(jnp.int32, (_ROWS, _LANES), 0)
    col_i = lax.broadcasted_iota(jnp.int32, (_ROWS, _LANES), 1)
    idx = row_i * _LANES + col_i
    idxf = idx.astype(f32)
    valid = idx < _N

    neg_one = jnp.full((_ROWS, _LANES), -1.0, f32)
    zeros = jnp.zeros((_ROWS, _LANES), f32)

    def body1(g, carry):
        run_max, mx1, my1, mx2, my2 = carry
        gx1 = gts_ref[g, 0]
        gy1 = gts_ref[g, 1]
        gx2 = gts_ref[g, 2]
        gy2 = gts_ref[g, 3]
        garea = (gx2 - gx1) * (gy2 - gy1)
        w = jnp.maximum(jnp.minimum(a3, gx2) - jnp.maximum(a1, gx1), 0.0)
        h = jnp.maximum(jnp.minimum(a4, gy2) - jnp.maximum(a2, gy1), 0.0)
        inter = w * h
        union = area_a + garea - inter
        iou = jnp.where(valid, inter / union, -1.0)
        iou_s[g] = iou
        cm_s[g] = jnp.max(iou, axis=0, keepdims=True)
        better = iou > run_max
        run_max = jnp.where(better, iou, run_max)
        mx1 = jnp.where(better, gx1, mx1)
        my1 = jnp.where(better, gy1, my1)
        mx2 = jnp.where(better, gx2, mx2)
        my2 = jnp.where(better, gy2, my2)
        return run_max, mx1, my1, mx2, my2

    init = (neg_one, zeros, zeros, zeros + 1.0, zeros + 1.0)
    run_max, mx1, my1, mx2, my2 = lax.fori_loop(0, _G, body1, init)

    cm = cm_s[...]
    colmax = jnp.max(cm, axis=2, keepdims=True)
    cm_s[...] = jnp.broadcast_to(colmax, (_G, 1, _LANES))

    def body2(g, carry):
        iou_g = iou_s[g]
        cmax_b = jnp.broadcast_to(cm_s[g], (_ROWS, _LANES))
        cand = jnp.where(iou_g == cmax_b, idxf, 3e7)
        pm_s[g] = jnp.min(cand, axis=0, keepdims=True)
        return carry

    lax.fori_loop(0, _G, body2, 0)
    pmv = pm_s[...]
    gidx = jnp.min(pmv, axis=2, keepdims=True)
    pm_s[...] = jnp.broadcast_to(gidx, (_G, 1, _LANES))

    def body3(g, is_gtmax_f):
        gb = jnp.broadcast_to(pm_s[g], (_ROWS, _LANES))
        return jnp.maximum(is_gtmax_f, (idxf == gb).astype(f32))

    is_gtmax = lax.fori_loop(0, _G, body3, zeros) > 0.0

    pos_mask = jnp.logical_and(
        valid, jnp.logical_or(run_max > _POS_THR, is_gtmax))
    neg_mask = jnp.logical_and(
        valid, jnp.logical_and(run_max < _NEG_THR,
                               jnp.logical_not(is_gtmax)))
    pos_f = pos_mask.astype(f32)
    neg_f = neg_mask.astype(f32)
    p_full = jnp.sum(pos_f)
    q_full = jnp.sum(neg_f)
    k_pos = jnp.minimum(p_full, float(_POS_UPPER))
    score_bug = p_full >= float(_POS_UPPER)
    neg_bound = jnp.where(p_full < float(_POS_UPPER), p_full,
                          float(_POS_UPPER))
    q_eff = jnp.minimum(neg_bound, q_full)

    bf16 = jnp.bfloat16
    t_lane = (lax.broadcasted_iota(jnp.int32, (_LANES, _LANES), 0)
              <= lax.broadcasted_iota(jnp.int32, (_LANES, _LANES), 1)
              ).astype(bf16)
    t_row = (lax.broadcasted_iota(jnp.int32, (_ROWS, _ROWS), 0)
             > lax.broadcasted_iota(jnp.int32, (_ROWS, _ROWS), 1)
             ).astype(bf16)

    def prefix_incl(m_f32):
        incl = jnp.dot(m_f32.astype(bf16), t_lane,
                       preferred_element_type=f32)
        rowtot = jnp.broadcast_to(
            jnp.sum(m_f32, axis=1, keepdims=True), (_ROWS, _LANES))
        offs = jnp.dot(t_row, rowtot.astype(bf16),
                       preferred_element_type=f32)
        return incl + offs

    sel_pos = jnp.logical_and(pos_mask, prefix_incl(pos_f) <= k_pos)
    sel_neg = jnp.logical_and(neg_mask, prefix_incl(neg_f) <= q_eff)
    sel_pos_f = sel_pos.astype(f32)

    s0 = s_ref[0]
    s1 = s_ref[1]
    sm = jnp.maximum(s0, s1)
    lse = sm + jnp.log(jnp.exp(s0 - sm) + jnp.exp(s1 - sm))
    logp0_score = s0 - lse
    logp1_score = s1 - lse
    am = jnp.maximum(jnp.maximum(a1, a2), jnp.maximum(a3, a4))
    alse = am + jnp.log(jnp.exp(a1 - am) + jnp.exp(a2 - am)
                        + jnp.exp(a3 - am) + jnp.exp(a4 - am))
    logp0_anch = a1 - alse
    pos_term = jnp.where(score_bug, logp0_anch, logp0_score)

    aw = a3 - a1
    ah = a4 - a2
    acx = a1 + aw * 0.5
    acy = a2 + ah * 0.5
    gw = mx2 - mx1
    gh = my2 - my1
    gcx = mx1 + gw * 0.5
    gcy = my1 + gh * 0.5
    tx = (gcx - acx) / aw
    ty = (gcy - acy) / ah
    tw = jnp.log(gw / aw)
    th = jnp.log(gh / ah)

    def sl1(pred, tgt):
        d = pred - tgt
        ad = jnp.abs(d)
        return jnp.where(ad < 1.0, 0.5 * d * d, ad - 0.5)

    reg_sum = (sl1(r_ref[0], tx) + sl1(r_ref[1], ty)
               + sl1(r_ref[2], tw) + sl1(r_ref[3], th))

    pos_cls = -jnp.sum(sel_pos_f * pos_term) / k_pos
    pos_reg = jnp.sum(sel_pos_f * reg_sum) / k_pos
    neg_cls = -jnp.sum(jnp.where(sel_neg, logp1_score, 0.0)) / q_eff
    out_ref[0, 0] = pos_cls + pos_reg + neg_cls


@jax.jit
def kernel(score_pred, reg_pred, anchors, gts):
    pad = _NPAD - _N
    a_t = jnp.pad(anchors.T, ((0, 0), (0, pad)))
    a_t = a_t.at[2:, _N:].set(1.0)
    s_t = jnp.pad(score_pred.T, ((0, 0), (0, pad)))
    r_t = jnp.pad(reg_pred.T, ((0, 0), (0, pad)))
    a3 = a_t.reshape(4, _ROWS, _LANES)
    s3 = s_t.reshape(2, _ROWS, _LANES)
    r3 = r_t.reshape(4, _ROWS, _LANES)

    out = pl.pallas_call(
        _atl_kernel,
        out_shape=jax.ShapeDtypeStruct((1, 1), jnp.float32),
        in_specs=[
            pl.BlockSpec(memory_space=pltpu.VMEM),
            pl.BlockSpec(memory_space=pltpu.VMEM),
            pl.BlockSpec(memory_space=pltpu.VMEM),
            pl.BlockSpec(memory_space=pltpu.SMEM),
        ],
        out_specs=pl.BlockSpec(memory_space=pltpu.SMEM),
        scratch_shapes=[
            pltpu.VMEM((_G, _ROWS, _LANES), jnp.float32),
            pltpu.VMEM((_G, 1, _LANES), jnp.float32),
            pltpu.VMEM((_G, 1, _LANES), jnp.float32),
        ],
    )(a3, s3, r3, gts)
    return out[0, 0]
